# baseline (speedup 1.0000x reference)
import functools

import jax
import jax.numpy as jnp
from jax import lax
from jax.experimental import pallas as pl
from jax.experimental.pallas import tpu as pltpu
from jax.experimental.pallas import tpu_sc as plsc

B = 4096
M = 64
K = 512
D = 64
IN_DIM = M * D

BT = 512

_NW = 32
_BPW = (B * M) // _NW
_CH = 128
_NCH = _BPW // _CH


def _argmax_body(x_ref, cb_ref, idx_ref):
    iota_f = lax.broadcasted_iota(jnp.int32, (BT, K), 1).astype(jnp.float32)
    for m in range(M):
        xm = x_ref[:, m * D:(m + 1) * D]
        cbm = cb_ref[m]
        dots = lax.dot_general(
            xm, cbm, (((1,), (1,)), ((), ())),
            preferred_element_type=jnp.float32)
        maxv = jnp.max(dots, axis=1, keepdims=True)
        idxf = jnp.min(jnp.where(dots == maxv, iota_f, jnp.float32(K)), axis=1)
        idx_ref[:, m:m + 1] = idxf.astype(jnp.int32)[:, None] + m * K


def _tc_argmax(x, codebooks):
    return pl.pallas_call(
        _argmax_body,
        grid=(B // BT,),
        in_specs=[
            pl.BlockSpec((BT, IN_DIM), lambda i: (i, 0)),
            pl.BlockSpec((M, K, D), lambda i: (0, 0, 0)),
        ],
        out_specs=pl.BlockSpec((BT, M), lambda i: (i, 0)),
        out_shape=jax.ShapeDtypeStruct((B, M), jnp.int32),
    )(x, codebooks)


_NB = 8


def _sc_gather_body(table_hbm, idx_hbm, out_hbm, idx_v, rows_v, gsem, wsem):
    wid = lax.axis_index("s") * 2 + lax.axis_index("c")
    base = wid * _NCH
    pltpu.sync_copy(idx_hbm.at[pl.ds(base, _NCH)], idx_v)

    def group(g, carry):
        @pl.when(g > 0)
        def _():
            for b in range(_NB):
                pltpu.make_async_copy(rows_v[b], out_hbm.at[pl.ds(0, _CH)],
                                      wsem).wait()

        cps = []
        for b in range(_NB):
            ci = g * _NB + b
            cps.append(pltpu.async_copy(table_hbm.at[idx_v.at[ci]],
                                        rows_v[b], gsem))
        for b in range(_NB):
            ci = g * _NB + b
            cps[b].wait()
            pltpu.async_copy(rows_v[b],
                             out_hbm.at[pl.ds((base + ci) * _CH, _CH)], wsem)
        return carry

    lax.fori_loop(0, _NCH // _NB, group, 0)
    for b in range(_NB):
        pltpu.make_async_copy(rows_v[b], out_hbm.at[pl.ds(0, _CH)], wsem).wait()


@functools.lru_cache(maxsize=None)
def _sc_gather_call():
    return functools.partial(
        pl.kernel,
        out_type=jax.ShapeDtypeStruct((B * M, D), jnp.float32),
        mesh=plsc.VectorSubcoreMesh(core_axis_name="c", subcore_axis_name="s"),
        scratch_types=[
            pltpu.VMEM((_NCH, _CH), jnp.int32),
            [pltpu.VMEM((_CH, D), jnp.float32) for _ in range(_NB)],
            pltpu.SemaphoreType.DMA,
            pltpu.SemaphoreType.DMA,
        ],
        compiler_params=pltpu.CompilerParams(use_tc_tiling_on_sc=False),
    )(_sc_gather_body)


def kernel(x, codebooks):
    flat_idx = _tc_argmax(x, codebooks).reshape(B * M // _CH, _CH)
    table = codebooks.reshape(M * K, D)
    rows = _sc_gather_call()(table, flat_idx)
    return rows.reshape(B, M * D)


# speedup vs baseline: 1.1715x; 1.1715x over previous
import functools

import jax
import jax.numpy as jnp
from jax import lax
from jax.experimental import pallas as pl
from jax.experimental.pallas import tpu as pltpu
from jax.experimental.pallas import tpu_sc as plsc

B = 4096
M = 64
K = 512
D = 64
IN_DIM = M * D

BT = 512

_NW = 32
_CH = 128
_NCB = B // _CH
_NB = 4


def _argmax_body(x_ref, cb_ref, idx_ref):
    iota_f = lax.broadcasted_iota(jnp.int32, (K, BT), 0).astype(jnp.float32)
    for m in range(M):
        xm = x_ref[:, m * D:(m + 1) * D]
        cbm = cb_ref[m]
        dots = lax.dot_general(
            cbm, xm, (((1,), (1,)), ((), ())),
            preferred_element_type=jnp.float32)
        maxv = jnp.max(dots, axis=0, keepdims=True)
        idxf = jnp.min(jnp.where(dots == maxv, iota_f, jnp.float32(K)),
                       axis=0)
        idx_ref[m:m + 1, :] = (idxf.astype(jnp.int32) + m * K)[None, :]


def _tc_argmax(x, codebooks):
    return pl.pallas_call(
        _argmax_body,
        grid=(B // BT,),
        in_specs=[
            pl.BlockSpec((BT, IN_DIM), lambda i: (i, 0)),
            pl.BlockSpec((M, K, D), lambda i: (0, 0, 0)),
        ],
        out_specs=pl.BlockSpec((M, BT), lambda i: (0, i)),
        out_shape=jax.ShapeDtypeStruct((M, B), jnp.int32),
    )(x, codebooks)


def _sc_gather_body(table_hbm, idx_hbm, out_hbm, idxa_v, idxb_v, rows_v,
                    gsem, wsem):
    wid = lax.axis_index("s") * 2 + lax.axis_index("c")
    pltpu.sync_copy(idx_hbm.at[2 * wid], idxa_v)
    pltpu.sync_copy(idx_hbm.at[2 * wid + 1], idxb_v)
    idx_planes = (idxa_v, idxb_v)

    def group(g, carry):
        @pl.when(g > 0)
        def _():
            for b in range(_NB):
                for h in range(2):
                    pltpu.make_async_copy(
                        rows_v[b][h],
                        out_hbm.at[pl.ds(0, _CH), 0, pl.ds(h * D, D)],
                        wsem).wait()

        cps = []
        for b in range(_NB):
            ci = g * _NB + b
            for h in range(2):
                cps.append(pltpu.async_copy(
                    table_hbm.at[idx_planes[h].at[ci]], rows_v[b][h], gsem))
        for b in range(_NB):
            ci = g * _NB + b
            for h in range(2):
                cps[2 * b + h].wait()
                pltpu.async_copy(
                    rows_v[b][h],
                    out_hbm.at[pl.ds(ci * _CH, _CH), wid, pl.ds(h * D, D)],
                    wsem)
        return carry

    lax.fori_loop(0, _NCB // _NB, group, 0)
    for b in range(_NB):
        for h in range(2):
            pltpu.make_async_copy(rows_v[b][h],
                                  out_hbm.at[pl.ds(0, _CH), 0, pl.ds(h * D, D)],
                                  wsem).wait()


@functools.lru_cache(maxsize=None)
def _sc_gather_call():
    return functools.partial(
        pl.kernel,
        out_type=jax.ShapeDtypeStruct((B, M // 2, 128), jnp.float32),
        mesh=plsc.VectorSubcoreMesh(core_axis_name="c", subcore_axis_name="s"),
        scratch_types=[
            pltpu.VMEM((_NCB, _CH), jnp.int32),
            pltpu.VMEM((_NCB, _CH), jnp.int32),
            [[pltpu.VMEM((_CH, D), jnp.float32) for _ in range(2)]
             for _ in range(_NB)],
            pltpu.SemaphoreType.DMA,
            pltpu.SemaphoreType.DMA,
        ],
        compiler_params=pltpu.CompilerParams(use_tc_tiling_on_sc=False),
    )(_sc_gather_body)


def kernel(x, codebooks):
    idx = _tc_argmax(x, codebooks).reshape(M, _NCB, _CH)
    table = codebooks.reshape(M * K, D)
    rows = _sc_gather_call()(table, idx)
    return rows.reshape(B, M * D)
